# baseline (speedup 1.0000x reference)
import functools

import jax
import jax.numpy as jnp
from jax import lax
from jax.experimental import pallas as pl
from jax.experimental.pallas import tpu as pltpu
from jax.experimental.pallas import tpu_sc as plsc

NUM_FIELDS = 26
VOCAB = 100000
EMB_DIM = 32
BATCH = 16384
NUM_DIM = 13
EPS = 1e-5

NC = 2
NS = 16
NW = NC * NS

LOOKUPS = BATCH * NUM_FIELDS
PER_W = LOOKUPS // NW
CHUNK = 1664
CHUNKS = PER_W // CHUNK
GATHER_LEN = 128
GATHERS = CHUNK // GATHER_LEN

BBLK = 512


def _prep_body(catT_ref, numT_ref, gamma_ref, beta_ref, fidxT_ref,
               numoutT_ref):
    idx = jnp.clip(catT_ref[...], 0, VOCAB - 1)
    field_off = jax.lax.broadcasted_iota(
        jnp.int32, (NUM_FIELDS, 1), 0) * VOCAB
    fidxT_ref[...] = idx + field_off
    x = numT_ref[...]
    mean = jnp.mean(x, axis=1, keepdims=True)
    var = jnp.mean((x - mean) * (x - mean), axis=1, keepdims=True)
    numoutT_ref[...] = (x - mean) * jax.lax.rsqrt(var + EPS) * gamma_ref[...] \
        + beta_ref[...]


def _prep(catT, numT, bn_gamma, bn_beta):
    return pl.pallas_call(
        _prep_body,
        out_shape=(
            jax.ShapeDtypeStruct((NUM_FIELDS, BATCH), jnp.int32),
            jax.ShapeDtypeStruct((NUM_DIM, BATCH), jnp.float32),
        ),
    )(catT, numT, bn_gamma.reshape(NUM_DIM, 1), bn_beta.reshape(NUM_DIM, 1))


def _gather_body(fidx_hbm, table_hbm, out_hbm, idx_v, rows_v, gsem):
    wid = lax.axis_index("s") * NC + lax.axis_index("c")
    for c in range(CHUNKS):
        base = (wid * CHUNKS + c) * CHUNK
        pltpu.sync_copy(fidx_hbm.at[pl.ds(base, CHUNK)], idx_v)
        handles = []
        for j in range(GATHERS):
            handles.append(pltpu.async_copy(
                table_hbm.at[idx_v.at[pl.ds(j * GATHER_LEN, GATHER_LEN)]],
                rows_v.at[pl.ds(j * GATHER_LEN, GATHER_LEN), :],
                gsem))
        for h in handles:
            h.wait()
        pltpu.sync_copy(rows_v, out_hbm.at[pl.ds(base, CHUNK), :])


@functools.partial(
    pl.kernel,
    mesh=plsc.VectorSubcoreMesh(core_axis_name="c", subcore_axis_name="s"),
    out_type=jax.ShapeDtypeStruct((LOOKUPS, EMB_DIM), jnp.float32),
    compiler_params=pltpu.CompilerParams(use_tc_tiling_on_sc=False),
    scratch_types=[
        pltpu.VMEM((CHUNK,), jnp.int32),
        pltpu.VMEM((CHUNK, EMB_DIM), jnp.float32),
        pltpu.SemaphoreType.DMA,
    ],
)
def _gather(fidx_hbm, table_hbm, out_hbm, idx_v, rows_v, gsem):
    _gather_body(fidx_hbm, table_hbm, out_hbm, idx_v, rows_v, gsem)


def _assemble_body(cat_ref, num_ref, out_ref):
    x = cat_ref[...]
    xt = jnp.swapaxes(x, 1, 2)
    out_ref[pl.ds(0, NUM_FIELDS * EMB_DIM), :] = xt.reshape(
        NUM_FIELDS * EMB_DIM, BBLK)
    out_ref[pl.ds(NUM_FIELDS * EMB_DIM, NUM_DIM), :] = num_ref[...]


def _assemble(catFM3, numoutT):
    return pl.pallas_call(
        _assemble_body,
        grid=(BATCH // BBLK,),
        in_specs=[
            pl.BlockSpec((NUM_FIELDS, BBLK, EMB_DIM), lambda i: (0, i, 0)),
            pl.BlockSpec((NUM_DIM, BBLK), lambda i: (0, i)),
        ],
        out_specs=pl.BlockSpec(
            (NUM_FIELDS * EMB_DIM + NUM_DIM, BBLK), lambda i: (0, i)),
        out_shape=jax.ShapeDtypeStruct(
            (NUM_FIELDS * EMB_DIM + NUM_DIM, BATCH), jnp.float32),
    )(catFM3, numoutT)


def kernel(categorical_inputs, numerical_inputs, tables, bn_gamma, bn_beta):
    catT = categorical_inputs.T
    numT = numerical_inputs.T
    fidxT, numoutT = _prep(catT, numT, bn_gamma, bn_beta)
    flat_table = tables.reshape(NUM_FIELDS * VOCAB, EMB_DIM)
    catFM = _gather(fidxT.reshape(LOOKUPS), flat_table)
    outT = _assemble(catFM.reshape(NUM_FIELDS, BATCH, EMB_DIM), numoutT)
    return outT.T


# speedup vs baseline: 1.3016x; 1.3016x over previous
import functools

import jax
import jax.numpy as jnp
from jax import lax
from jax.experimental import pallas as pl
from jax.experimental.pallas import tpu as pltpu
from jax.experimental.pallas import tpu_sc as plsc

NUM_FIELDS = 26
VOCAB = 100000
EMB_DIM = 32
BATCH = 16384
NUM_DIM = 13
EPS = 1e-5

NC = 2
NS = 16
NW = NC * NS

NGROUP = (NUM_FIELDS + 3) // 4
GROW = 4 * EMB_DIM

LOOKUPS = BATCH * NUM_FIELDS
CHUNK = BATCH // NW
GATHER_LEN = 128
GATHERS = CHUNK // GATHER_LEN

VBLK = 1024
VPAD = 100352
BBLK = 512


def _regroup_body(tabT_ref, out_ref):
    x = tabT_ref[...]
    parts = [jnp.swapaxes(x[q], 0, 1) for q in range(4)]
    out_ref[...] = jnp.concatenate(parts, axis=1)


def _regroup(tablesT):
    return pl.pallas_call(
        _regroup_body,
        grid=(NGROUP, VPAD // VBLK),
        in_specs=[
            pl.BlockSpec((4, EMB_DIM, VBLK), lambda g, j: (g, 0, j)),
        ],
        out_specs=pl.BlockSpec(
            (VBLK, GROW), lambda g, j: (g * (VPAD // VBLK) + j, 0)),
        out_shape=jax.ShapeDtypeStruct((NGROUP * VPAD, GROW), jnp.float32),
    )(tablesT)


def _prep_body(catT_ref, numT_ref, gamma_ref, beta_ref, fidxT_ref,
               numoutT_ref):
    idx = jnp.clip(catT_ref[...], 0, VOCAB - 1)
    group = jax.lax.broadcasted_iota(
        jnp.int32, (NUM_FIELDS, 1), 0) // 4
    flat = (idx + group * VPAD).astype(jnp.float32)
    bb = jax.lax.broadcasted_iota(jnp.int32, (BBLK, BBLK), 0)
    pp = jax.lax.broadcasted_iota(jnp.int32, (BBLK, BBLK), 1)
    perm = (bb == (pp % 4) * (BBLK // 4) + pp // 4).astype(jnp.float32)
    v = jnp.dot(flat.reshape(NUM_FIELDS * (BATCH // BBLK), BBLK), perm,
                preferred_element_type=jnp.float32,
                precision=jax.lax.Precision.HIGHEST)
    fidxT_ref[...] = v.reshape(NUM_FIELDS, BATCH).astype(jnp.int32)
    x = numT_ref[...]
    mean = jnp.mean(x, axis=1, keepdims=True)
    var = jnp.mean((x - mean) * (x - mean), axis=1, keepdims=True)
    numoutT_ref[...] = (x - mean) * jax.lax.rsqrt(var + EPS) * gamma_ref[...] \
        + beta_ref[...]


def _prep(catT, numT, bn_gamma, bn_beta):
    return pl.pallas_call(
        _prep_body,
        out_shape=(
            jax.ShapeDtypeStruct((NUM_FIELDS, BATCH), jnp.int32),
            jax.ShapeDtypeStruct((NUM_DIM, BATCH), jnp.float32),
        ),
    )(catT, numT, bn_gamma.reshape(NUM_DIM, 1), bn_beta.reshape(NUM_DIM, 1))


def _gather_body(fidx_hbm, table_hbm, out_hbm, idx_v, rows_v, gsem):
    wid = lax.axis_index("s") * NC + lax.axis_index("c")
    for k in range(NUM_FIELDS):
        q = k % 4
        base = k * BATCH + wid * CHUNK
        pltpu.sync_copy(fidx_hbm.at[pl.ds(base, CHUNK)], idx_v)
        handles = []
        for j in range(GATHERS):
            handles.append(pltpu.async_copy(
                table_hbm.at[idx_v.at[pl.ds(j * GATHER_LEN, GATHER_LEN)]],
                rows_v.at[pl.ds(j * GATHER_LEN, GATHER_LEN), :],
                gsem))
        for h in handles:
            h.wait()
        pltpu.sync_copy(rows_v.at[:, pl.ds(q * EMB_DIM, EMB_DIM)],
                        out_hbm.at[pl.ds(base, CHUNK), :])


def _gather(fidx, gtable):
    k = functools.partial(
        pl.kernel,
        mesh=plsc.VectorSubcoreMesh(core_axis_name="c", subcore_axis_name="s"),
        out_type=jax.ShapeDtypeStruct((LOOKUPS, EMB_DIM), jnp.float32),
        compiler_params=pltpu.CompilerParams(use_tc_tiling_on_sc=False),
        scratch_types=[
            pltpu.VMEM((CHUNK,), jnp.int32),
            pltpu.VMEM((CHUNK, GROW), jnp.float32),
            pltpu.SemaphoreType.DMA,
        ],
    )(_gather_body)
    return k(fidx, gtable)


def _assemble_body(cat_ref, num_ref, out_ref):
    x = cat_ref[...]
    rows = []
    for f in range(NUM_FIELDS):
        parts = [
            jnp.swapaxes(x[f, :, q * EMB_DIM:(q + 1) * EMB_DIM], 0, 1)
            for q in range(4)
        ]
        rows.append(jnp.concatenate(parts, axis=1))
    out_ref[pl.ds(0, NUM_FIELDS * EMB_DIM), :] = jnp.concatenate(rows, axis=0)
    out_ref[pl.ds(NUM_FIELDS * EMB_DIM, NUM_DIM), :] = num_ref[...]


def _assemble(catP, numoutT):
    return pl.pallas_call(
        _assemble_body,
        grid=(BATCH // BBLK,),
        in_specs=[
            pl.BlockSpec((NUM_FIELDS, BBLK // 4, GROW), lambda i: (0, i, 0)),
            pl.BlockSpec((NUM_DIM, BBLK), lambda i: (0, i)),
        ],
        out_specs=pl.BlockSpec(
            (NUM_FIELDS * EMB_DIM + NUM_DIM, BBLK), lambda i: (0, i)),
        out_shape=jax.ShapeDtypeStruct(
            (NUM_FIELDS * EMB_DIM + NUM_DIM, BATCH), jnp.float32),
    )(catP, numoutT)


def kernel(categorical_inputs, numerical_inputs, tables, bn_gamma, bn_beta):
    catT = categorical_inputs.T
    numT = numerical_inputs.T
    tablesT = tables.transpose(0, 2, 1)
    gtable = _regroup(tablesT)
    fidxT, numoutT = _prep(catT, numT, bn_gamma, bn_beta)
    catFM = _gather(fidxT.reshape(LOOKUPS), gtable)
    catP = catFM.reshape(LOOKUPS * EMB_DIM).reshape(
        NUM_FIELDS, BATCH // 4, GROW)
    outT = _assemble(catP, numoutT)
    return outT.T


# speedup vs baseline: 1.7252x; 1.3255x over previous
import functools

import jax
import jax.numpy as jnp
from jax import lax
from jax.experimental import pallas as pl
from jax.experimental.pallas import tpu as pltpu
from jax.experimental.pallas import tpu_sc as plsc

NUM_FIELDS = 26
VOCAB = 100000
EMB_DIM = 32
BATCH = 16384
NUM_DIM = 13
EPS = 1e-5

NC = 2
NS = 16
NW = NC * NS

NGROUP = (NUM_FIELDS + 3) // 4
GROW = 4 * EMB_DIM

LOOKUPS = BATCH * NUM_FIELDS
CHUNK = BATCH // NW
GATHER_LEN = 128
GATHERS = CHUNK // GATHER_LEN

VBLK = 1024
VPAD = 100352
BBLK = 512


def _regroup_body(tabT_ref, out_ref):
    x = tabT_ref[...]
    out_ref[...] = jnp.swapaxes(x.reshape(GROW, VBLK), 0, 1)


def _regroup(tablesT):
    return pl.pallas_call(
        _regroup_body,
        grid=(NGROUP, VPAD // VBLK),
        in_specs=[
            pl.BlockSpec((4, EMB_DIM, VBLK), lambda g, j: (g, 0, j)),
        ],
        out_specs=pl.BlockSpec(
            (VBLK, GROW), lambda g, j: (g * (VPAD // VBLK) + j, 0)),
        out_shape=jax.ShapeDtypeStruct((NGROUP * VPAD, GROW), jnp.float32),
    )(tablesT)


def _prep_body(catT_ref, numT_ref, gamma_ref, beta_ref, fidxT_ref,
               numoutT_ref):
    idx = jnp.clip(catT_ref[...], 0, VOCAB - 1)
    group = jax.lax.broadcasted_iota(
        jnp.int32, (NUM_FIELDS, 1), 0) // 4
    flat = (idx + group * VPAD).astype(jnp.float32)
    bb = jax.lax.broadcasted_iota(jnp.int32, (BBLK, BBLK), 0)
    pp = jax.lax.broadcasted_iota(jnp.int32, (BBLK, BBLK), 1)
    perm = (bb == (pp % 4) * (BBLK // 4) + pp // 4).astype(jnp.float32)
    v = jnp.dot(flat.reshape(NUM_FIELDS * (BATCH // BBLK), BBLK), perm,
                preferred_element_type=jnp.float32,
                precision=jax.lax.Precision.HIGHEST)
    fidxT_ref[...] = v.reshape(NUM_FIELDS, BATCH).astype(jnp.int32)
    x = numT_ref[...]
    mean = jnp.mean(x, axis=1, keepdims=True)
    var = jnp.mean((x - mean) * (x - mean), axis=1, keepdims=True)
    numoutT_ref[...] = (x - mean) * jax.lax.rsqrt(var + EPS) * gamma_ref[...] \
        + beta_ref[...]


def _prep(catT, numT, bn_gamma, bn_beta):
    return pl.pallas_call(
        _prep_body,
        out_shape=(
            jax.ShapeDtypeStruct((NUM_FIELDS, BATCH), jnp.int32),
            jax.ShapeDtypeStruct((NUM_DIM, BATCH), jnp.float32),
        ),
    )(catT, numT, bn_gamma.reshape(NUM_DIM, 1), bn_beta.reshape(NUM_DIM, 1))


def _gather_body(fidx_hbm, table_hbm, out_hbm, idx_v, rows_v, gsem):
    wid = lax.axis_index("s") * NC + lax.axis_index("c")
    for k in range(NUM_FIELDS):
        q = k % 4
        base = k * BATCH + wid * CHUNK
        pltpu.sync_copy(fidx_hbm.at[pl.ds(base, CHUNK)], idx_v)
        handles = []
        for j in range(GATHERS):
            handles.append(pltpu.async_copy(
                table_hbm.at[idx_v.at[pl.ds(j * GATHER_LEN, GATHER_LEN)]],
                rows_v.at[pl.ds(j * GATHER_LEN, GATHER_LEN), :],
                gsem))
        for h in handles:
            h.wait()
        pltpu.sync_copy(rows_v.at[:, pl.ds(q * EMB_DIM, EMB_DIM)],
                        out_hbm.at[pl.ds(base, CHUNK), :])


def _gather(fidx, gtable):
    k = functools.partial(
        pl.kernel,
        mesh=plsc.VectorSubcoreMesh(core_axis_name="c", subcore_axis_name="s"),
        out_type=jax.ShapeDtypeStruct((LOOKUPS, EMB_DIM), jnp.float32),
        compiler_params=pltpu.CompilerParams(use_tc_tiling_on_sc=False),
        scratch_types=[
            pltpu.VMEM((CHUNK,), jnp.int32),
            pltpu.VMEM((CHUNK, GROW), jnp.float32),
            pltpu.SemaphoreType.DMA,
        ],
    )(_gather_body)
    return k(fidx, gtable)


def _assemble_body(cat_ref, num_ref, out_ref):
    x = cat_ref[...]
    rows = []
    for f in range(NUM_FIELDS):
        parts = [
            jnp.swapaxes(x[f, :, q * EMB_DIM:(q + 1) * EMB_DIM], 0, 1)
            for q in range(4)
        ]
        rows.append(jnp.concatenate(parts, axis=1))
    out_ref[pl.ds(0, NUM_FIELDS * EMB_DIM), :] = jnp.concatenate(rows, axis=0)
    out_ref[pl.ds(NUM_FIELDS * EMB_DIM, NUM_DIM), :] = num_ref[...]


def _assemble(catP, numoutT):
    return pl.pallas_call(
        _assemble_body,
        grid=(BATCH // BBLK,),
        in_specs=[
            pl.BlockSpec((NUM_FIELDS, BBLK // 4, GROW), lambda i: (0, i, 0)),
            pl.BlockSpec((NUM_DIM, BBLK), lambda i: (0, i)),
        ],
        out_specs=pl.BlockSpec(
            (NUM_FIELDS * EMB_DIM + NUM_DIM, BBLK), lambda i: (0, i)),
        out_shape=jax.ShapeDtypeStruct(
            (NUM_FIELDS * EMB_DIM + NUM_DIM, BATCH), jnp.float32),
    )(catP, numoutT)


def kernel(categorical_inputs, numerical_inputs, tables, bn_gamma, bn_beta):
    catT = categorical_inputs.T
    numT = numerical_inputs.T
    tablesT = tables.transpose(0, 2, 1)
    gtable = _regroup(tablesT)
    fidxT, numoutT = _prep(catT, numT, bn_gamma, bn_beta)
    catFM = _gather(fidxT.reshape(LOOKUPS), gtable)
    catP = catFM.reshape(LOOKUPS * EMB_DIM).reshape(
        NUM_FIELDS, BATCH // 4, GROW)
    outT = _assemble(catP, numoutT)
    return outT.T


# speedup vs baseline: 2.3351x; 1.3535x over previous
import functools

import jax
import jax.numpy as jnp
from jax import lax
from jax.experimental import pallas as pl
from jax.experimental.pallas import tpu as pltpu
from jax.experimental.pallas import tpu_sc as plsc

NUM_FIELDS = 26
VOCAB = 100000
EMB_DIM = 32
BATCH = 16384
NUM_DIM = 13
EPS = 1e-5

NC = 2
NS = 16
NW = NC * NS

NGROUP = (NUM_FIELDS + 3) // 4
GROW = 4 * EMB_DIM

LOOKUPS = BATCH * NUM_FIELDS
CHUNK = BATCH // NW
GATHER_LEN = 128
GATHERS = CHUNK // GATHER_LEN

VBLK = 2048
VPAD = 100352
BBLK = 512


def _regroup_body(tabT_ref, out_ref):
    x = tabT_ref[...]
    out_ref[...] = jnp.swapaxes(x.reshape(GROW, VBLK), 0, 1)


def _regroup(tablesT):
    return pl.pallas_call(
        _regroup_body,
        grid=(NGROUP, VPAD // VBLK),
        in_specs=[
            pl.BlockSpec((4, EMB_DIM, VBLK), lambda g, j: (g, 0, j)),
        ],
        out_specs=pl.BlockSpec(
            (VBLK, GROW), lambda g, j: (g * (VPAD // VBLK) + j, 0)),
        out_shape=jax.ShapeDtypeStruct((NGROUP * VPAD, GROW), jnp.float32),
    )(tablesT)


def _prep_body(catT_ref, numT_ref, gamma_ref, beta_ref, fidxT_ref,
               numoutT_ref):
    idx = jnp.clip(catT_ref[...], 0, VOCAB - 1)
    group = jax.lax.broadcasted_iota(
        jnp.int32, (NUM_FIELDS, 1), 0) // 4
    flat = (idx + group * VPAD).astype(jnp.float32)
    bb = jax.lax.broadcasted_iota(jnp.int32, (BBLK, BBLK), 0)
    pp = jax.lax.broadcasted_iota(jnp.int32, (BBLK, BBLK), 1)
    perm = (bb == (pp % 4) * (BBLK // 4) + pp // 4).astype(jnp.float32)
    v = jnp.dot(flat.reshape(NUM_FIELDS * (BATCH // BBLK), BBLK), perm,
                preferred_element_type=jnp.float32,
                precision=jax.lax.Precision.HIGHEST)
    fidxT_ref[...] = v.reshape(NUM_FIELDS, BATCH).astype(jnp.int32)
    x = numT_ref[...]
    mean = jnp.mean(x, axis=1, keepdims=True)
    var = jnp.mean((x - mean) * (x - mean), axis=1, keepdims=True)
    numoutT_ref[...] = (x - mean) * jax.lax.rsqrt(var + EPS) * gamma_ref[...] \
        + beta_ref[...]


def _prep(catT, numT, bn_gamma, bn_beta):
    return pl.pallas_call(
        _prep_body,
        out_shape=(
            jax.ShapeDtypeStruct((NUM_FIELDS, BATCH), jnp.int32),
            jax.ShapeDtypeStruct((NUM_DIM, BATCH), jnp.float32),
        ),
    )(catT, numT, bn_gamma.reshape(NUM_DIM, 1), bn_beta.reshape(NUM_DIM, 1))


def _gather_body(fidx_hbm, table_hbm, out_hbm, idx_v, rows_v, gsem):
    wid = lax.axis_index("s") * NC + lax.axis_index("c")
    for k in range(NUM_FIELDS):
        q = k % 4
        base = k * BATCH + wid * CHUNK
        pltpu.sync_copy(fidx_hbm.at[pl.ds(base, CHUNK)], idx_v)
        handles = []
        for j in range(GATHERS):
            handles.append(pltpu.async_copy(
                table_hbm.at[idx_v.at[pl.ds(j * GATHER_LEN, GATHER_LEN)]],
                rows_v.at[pl.ds(j * GATHER_LEN, GATHER_LEN), :],
                gsem))
        for h in handles:
            h.wait()
        pltpu.sync_copy(rows_v.at[:, pl.ds(q * EMB_DIM, EMB_DIM)],
                        out_hbm.at[pl.ds(base, CHUNK), :])


def _gather(fidx, gtable):
    k = functools.partial(
        pl.kernel,
        mesh=plsc.VectorSubcoreMesh(core_axis_name="c", subcore_axis_name="s"),
        out_type=jax.ShapeDtypeStruct((LOOKUPS, EMB_DIM), jnp.float32),
        compiler_params=pltpu.CompilerParams(use_tc_tiling_on_sc=False),
        scratch_types=[
            pltpu.VMEM((CHUNK,), jnp.int32),
            pltpu.VMEM((CHUNK, GROW), jnp.float32),
            pltpu.SemaphoreType.DMA,
        ],
    )(_gather_body)
    return k(fidx, gtable)


def _assemble_body(cat_ref, num_ref, out_ref):
    x = cat_ref[...]
    rows = []
    for f in range(NUM_FIELDS):
        t = jnp.swapaxes(x[f], 0, 1)
        t = t.reshape(4, EMB_DIM, BBLK // 4)
        rows.append(jnp.swapaxes(t, 0, 1).reshape(EMB_DIM, BBLK))
    out_ref[pl.ds(0, NUM_FIELDS * EMB_DIM), :] = jnp.concatenate(rows, axis=0)
    out_ref[pl.ds(NUM_FIELDS * EMB_DIM, NUM_DIM), :] = num_ref[...]


def _assemble(catP, numoutT):
    return pl.pallas_call(
        _assemble_body,
        grid=(BATCH // BBLK,),
        in_specs=[
            pl.BlockSpec((NUM_FIELDS, BBLK // 4, GROW), lambda i: (0, i, 0)),
            pl.BlockSpec((NUM_DIM, BBLK), lambda i: (0, i)),
        ],
        out_specs=pl.BlockSpec(
            (NUM_FIELDS * EMB_DIM + NUM_DIM, BBLK), lambda i: (0, i)),
        out_shape=jax.ShapeDtypeStruct(
            (NUM_FIELDS * EMB_DIM + NUM_DIM, BATCH), jnp.float32),
    )(catP, numoutT)


def kernel(categorical_inputs, numerical_inputs, tables, bn_gamma, bn_beta):
    catT = categorical_inputs.T
    numT = numerical_inputs.T
    tablesT = tables.transpose(0, 2, 1)
    gtable = _regroup(tablesT)
    fidxT, numoutT = _prep(catT, numT, bn_gamma, bn_beta)
    catFM = _gather(fidxT.reshape(LOOKUPS), gtable)
    catP = catFM.reshape(LOOKUPS * EMB_DIM).reshape(
        NUM_FIELDS, BATCH // 4, GROW)
    outT = _assemble(catP, numoutT)
    return outT.T
